# baseline (speedup 1.0000x reference)
import functools

import jax
import jax.numpy as jnp
from jax import lax
from jax.experimental import pallas as pl
from jax.experimental.pallas import tpu as pltpu
from jax.experimental.pallas import tpu_sc as plsc

N = 25000
D = 128
EDGES = 312500
L = 16
NC, NS = 2, 16
NW = NC * NS
SEG_P = 25088
SEG_T = SEG_P // NW
PAD_DST = 25024
TPE = 9984
EP = NW * TPE
CH = 128
NCH_A = TPE // CH
CHL = 1248
NCHL = TPE // CHL
TPE_E = EP // NS
NCH_E = TPE_E // CH
ROWS_T = SEG_P // NS
DH = D // 2
DQ = D // 4
NEG = -3.0e38

f32 = jnp.float32
i32 = jnp.int32


def _mesh():
    return plsc.VectorSubcoreMesh(core_axis_name="c", subcore_axis_name="s")


_SC_PARAMS = pltpu.CompilerParams(needs_layout_passes=False)
_SC_PARAMS_SCT = pltpu.CompilerParams(needs_layout_passes=False,
                                      use_tc_tiling_on_sc=False)


def _colsum(x):
    blk = 1000
    nblk = N // blk

    def k(x_ref, o_ref):
        i = pl.program_id(0)
        part = jnp.sum(x_ref[...], axis=0, keepdims=True)
        pb = jnp.broadcast_to(part, (8, D))

        @pl.when(i == 0)
        def _():
            o_ref[...] = pb

        @pl.when(i != 0)
        def _():
            o_ref[...] = o_ref[...] + pb

    return pl.pallas_call(
        k,
        grid=(nblk,),
        in_specs=[pl.BlockSpec((blk, D), lambda i: (i, 0))],
        out_specs=pl.BlockSpec((8, D), lambda i: (0, 0)),
        out_shape=jax.ShapeDtypeStruct((8, D), f32),
    )(x)


def _proj(x, emb, s8, wo, bo, wg, bg, wu, bu):
    blk = 1000
    nblk = N // blk

    def k(x_ref, emb_ref, s_ref, wo_ref, bo_ref, wg_ref, bg_ref, wu_ref,
          bu_ref, h_ref, ehp_ref):
        s = s_ref[0:1, :]
        g = (jnp.dot(s, wg_ref[...], preferred_element_type=f32)
             + bg_ref[...] + bo_ref[...])
        h_ref[...] = jnp.dot(x_ref[...], wo_ref[...],
                             preferred_element_type=f32) + g
        ehp_ref[...] = jnp.dot(emb_ref[...], wu_ref[...],
                               preferred_element_type=f32) + bu_ref[...]

    full = lambda shape: pl.BlockSpec(shape, lambda i: (0,) * len(shape))
    return pl.pallas_call(
        k,
        grid=(nblk,),
        in_specs=[
            pl.BlockSpec((blk, D), lambda i: (i, 0)),
            pl.BlockSpec((blk, D), lambda i: (i, 0)),
            full((8, D)),
            full((D, D)), full((1, D)),
            full((D, D)), full((1, D)),
            full((D, D)), full((1, D)),
        ],
        out_specs=[
            pl.BlockSpec((blk, D), lambda i: (i, 0)),
            pl.BlockSpec((blk, D), lambda i: (i, 0)),
        ],
        out_shape=[
            jax.ShapeDtypeStruct((N, D), f32),
            jax.ShapeDtypeStruct((N, D), f32),
        ],
    )(x, emb, s8, wo, bo.reshape(1, D), wg, bg.reshape(1, D), wu,
      bu.reshape(1, D))


def _gather_body(h_hbm, ehp_hbm, src_hbm, npid_hbm, hh_hbm,
                 hb0, hb1, si0, si1, ni0, ni1,
                 isem0, isem1, gsem0, gsem1, asem0, asem1, wsem0, wsem1):
    wid = lax.axis_index("s") * NC + lax.axis_index("c")
    base_t = wid * TPE
    hbs, sis, nis = (hb0, hb1), (si0, si1), (ni0, ni1)
    isems, gsems = (isem0, isem1), (gsem0, gsem1)
    asems, wsems = (asem0, asem1), (wsem0, wsem1)

    def issue_idx(i, b):
        base = base_t + i * CH
        pltpu.async_copy(src_hbm.at[pl.ds(base, CH)], sis[b], isems[b])
        pltpu.async_copy(npid_hbm.at[pl.ds(base, CH)], nis[b], isems[b])

    def wait_idx(b):
        pltpu.make_async_copy(src_hbm.at[pl.ds(0, CH)], sis[b],
                              isems[b]).wait()
        pltpu.make_async_copy(npid_hbm.at[pl.ds(0, CH)], nis[b],
                              isems[b]).wait()

    def wait_wb(b):
        pltpu.make_async_copy(hbs[b], hh_hbm.at[pl.ds(base_t, CH)],
                              wsems[b]).wait()

    issue_idx(0, 0)
    issue_idx(1, 1)
    wait_idx(0)
    pltpu.async_copy(h_hbm.at[sis[0]], hbs[0], gsems[0])

    NP = NCH_A // 2

    def pair(p, _):
        for b in range(2):
            i = p * 2 + b
            base = base_t + i * CH
            bn = 1 - b
            if b == 0:
                wait_idx(bn)

                @pl.when(p > 0)
                def _():
                    wait_wb(bn)
                pltpu.async_copy(h_hbm.at[sis[bn]], hbs[bn], gsems[bn])
            else:
                @pl.when(p < NP - 1)
                def _():
                    wait_idx(bn)
                    wait_wb(bn)
                    pltpu.async_copy(h_hbm.at[sis[bn]], hbs[bn], gsems[bn])
            pltpu.make_async_copy(h_hbm.at[sis[b]], hbs[b], gsems[b]).wait()
            pltpu.async_copy(ehp_hbm.at[nis[b]], hbs[b], asems[b], add=True)
            pltpu.make_async_copy(ehp_hbm.at[nis[b]], hbs[b],
                                  asems[b]).wait()

            @pl.when(p < NP - 1)
            def _():
                issue_idx(i + 2, b)
            pltpu.async_copy(hbs[b], hh_hbm.at[pl.ds(base, CH)], wsems[b])
        return 0

    lax.fori_loop(0, NP, pair, 0)
    wait_wb(0)
    wait_wb(1)


def _gather_call(h, ehp, srcp, npidp):
    kfn = pl.kernel(
        _gather_body,
        out_type=jax.ShapeDtypeStruct((EP, D), f32),
        mesh=_mesh(),
        compiler_params=_SC_PARAMS,
        scratch_types=(
            [pltpu.VMEM((CH, D), f32)] * 2
            + [pltpu.VMEM((CH,), i32)] * 4
            + [pltpu.SemaphoreType.DMA] * 8
        ),
    )
    return kfn(h, ehp, srcp, npidp)


def _attdot(hh, att_w):
    blk = 1024
    nblk = EP // blk

    def k(hh_ref, w_ref, a_ref):
        v = hh_ref[...]
        lv = jnp.maximum(v, v * 0.01)
        a_ref[...] = jnp.dot(lv, w_ref[...], preferred_element_type=f32)

    return pl.pallas_call(
        k,
        grid=(nblk,),
        in_specs=[
            pl.BlockSpec((blk, D), lambda i: (i, 0)),
            pl.BlockSpec((D, 1), lambda i: (0, 0)),
        ],
        out_specs=pl.BlockSpec((blk, 1), lambda i: (i, 0)),
        out_shape=jax.ShapeDtypeStruct((EP, 1), f32),
    )(hh, att_w)


def _locmax_body(a_hbm, dst_hbm, locmax_hbm,
                 av0, av1, dv0, dv1, lm, chk, isem0, isem1):
    wid = lax.axis_index("s") * NC + lax.axis_index("c")
    base_t = wid * TPE
    iota = lax.iota(i32, L)
    masks = [iota == l for l in range(L)]
    avs, dvs, isems = (av0, av1), (dv0, dv1), (isem0, isem1)

    def issue(i, b):
        base = base_t + i * CHL
        pltpu.async_copy(a_hbm.at[pl.ds(base, CHL)], avs[b], isems[b])
        pltpu.async_copy(dst_hbm.at[pl.ds(base, CHL)], dvs[b], isems[b])

    def wait(b):
        pltpu.make_async_copy(a_hbm.at[pl.ds(0, CHL)], avs[b],
                              isems[b]).wait()
        pltpu.make_async_copy(dst_hbm.at[pl.ds(0, CHL)], dvs[b],
                              isems[b]).wait()

    neg = jnp.full((L,), NEG, f32)

    def init(iblk, _):
        plsc.store_scatter(lm, [iblk * L + iota], neg)
        return 0

    issue(0, 0)
    issue(1, 1)
    lax.fori_loop(0, SEG_P // L, init, 0)

    def pair(p, _):
        for b in range(2):
            i = p * 2 + b
            wait(b)
            av, dv = avs[b], dvs[b]

            def grp(g, _):
                off = g * L + iota
                a16 = plsc.load_gather(av, [off])
                d16 = plsc.load_gather(dv, [off])
                plsc.store_scatter(chk, [d16], iota)
                rb = plsc.load_gather(chk, [d16])
                nodup = jnp.all(rb == iota)

                @pl.when(nodup)
                def _():
                    cur = plsc.load_gather(lm, [d16])
                    plsc.store_scatter(lm, [d16], jnp.maximum(cur, a16))

                @pl.when(jnp.logical_not(nodup))
                def _():
                    for l in range(L):
                        cur = plsc.load_gather(lm, [d16])
                        plsc.store_scatter(lm, [d16],
                                           jnp.maximum(cur, a16),
                                           mask=masks[l])
                return 0

            lax.fori_loop(0, CHL // L, grp, 0)

            @pl.when(p * 2 + b + 2 < NCHL)
            def _():
                issue(i + 2, b)
        return 0

    lax.fori_loop(0, NCHL // 2, pair, 0)
    pltpu.sync_copy(lm, locmax_hbm.at[pl.ds(wid * SEG_P, SEG_P)])


def _locmax_call(a, dstp):
    kfn = pl.kernel(
        _locmax_body,
        out_type=jax.ShapeDtypeStruct((NW * SEG_P,), f32),
        mesh=_mesh(),
        compiler_params=_SC_PARAMS,
        scratch_types=[
            pltpu.VMEM((CHL,), f32),
            pltpu.VMEM((CHL,), f32),
            pltpu.VMEM((CHL,), i32),
            pltpu.VMEM((CHL,), i32),
            pltpu.VMEM((SEG_P,), f32),
            pltpu.VMEM((SEG_P,), i32),
            pltpu.SemaphoreType.DMA,
            pltpu.SemaphoreType.DMA,
        ],
    )
    return kfn(a, dstp)


def _combine_body(is_max, loc_hbm, out_hbm, acc, buf0, buf1, sem0, sem1):
    wid = lax.axis_index("s") * NC + lax.axis_index("c")
    seg0 = wid * SEG_T
    bufs, sems = (buf0, buf1), (sem0, sem1)

    def issue(r, b):
        pltpu.async_copy(loc_hbm.at[pl.ds(r * SEG_P + seg0, SEG_T)],
                         bufs[b], sems[b])

    def wait(b):
        pltpu.make_async_copy(loc_hbm.at[pl.ds(seg0, SEG_T)], bufs[b],
                              sems[b]).wait()

    issue(0, 0)
    issue(1, 1)
    fill = jnp.full((L,), NEG if is_max else 0.0, f32)
    for ib in range(SEG_T // L):
        acc[pl.ds(ib * L, L)] = fill

    def pair(p, _):
        for b in range(2):
            r = p * 2 + b
            wait(b)
            for ib in range(SEG_T // L):
                sl = pl.ds(ib * L, L)
                x = acc[sl]
                y = bufs[b][sl]
                acc[sl] = jnp.maximum(x, y) if is_max else x + y

            @pl.when(r + 2 < NW)
            def _():
                issue(r + 2, b)
        return 0

    lax.fori_loop(0, NW // 2, pair, 0)
    pltpu.sync_copy(acc, out_hbm.at[pl.ds(seg0, SEG_T)])


def _combine_call(loc, is_max):
    kfn = pl.kernel(
        functools.partial(_combine_body, is_max),
        out_type=jax.ShapeDtypeStruct((SEG_P,), f32),
        mesh=_mesh(),
        compiler_params=_SC_PARAMS,
        scratch_types=[
            pltpu.VMEM((SEG_T,), f32),
            pltpu.VMEM((SEG_T,), f32),
            pltpu.VMEM((SEG_T,), f32),
            pltpu.SemaphoreType.DMA,
            pltpu.SemaphoreType.DMA,
        ],
    )
    return kfn(loc)


def _expsum_body(a_hbm, dst_hbm, m_hbm, e_hbm, locsum_hbm,
                 av0, av1, dv0, dv1, eb0, eb1, mloc, ls,
                 isem0, isem1, wsem0, wsem1):
    wid = lax.axis_index("s") * NC + lax.axis_index("c")
    base_t = wid * TPE
    iota = lax.iota(i32, L)
    masks = [iota == l for l in range(L)]
    avs, dvs, ebs = (av0, av1), (dv0, dv1), (eb0, eb1)
    isems, wsems = (isem0, isem1), (wsem0, wsem1)

    def issue(i, b):
        base = base_t + i * CHL
        pltpu.async_copy(a_hbm.at[pl.ds(base, CHL)], avs[b], isems[b])
        pltpu.async_copy(dst_hbm.at[pl.ds(base, CHL)], dvs[b], isems[b])

    def wait(b):
        pltpu.make_async_copy(a_hbm.at[pl.ds(0, CHL)], avs[b],
                              isems[b]).wait()
        pltpu.make_async_copy(dst_hbm.at[pl.ds(0, CHL)], dvs[b],
                              isems[b]).wait()

    def wait_wb(b):
        pltpu.make_async_copy(ebs[b], e_hbm.at[pl.ds(base_t, CHL)],
                              wsems[b]).wait()

    issue(0, 0)
    issue(1, 1)
    pltpu.sync_copy(m_hbm, mloc)

    zero = jnp.zeros((L,), f32)

    def init(iblk, _):
        plsc.store_scatter(ls, [iblk * L + iota], zero)
        return 0

    lax.fori_loop(0, SEG_P // L, init, 0)

    def pair(p, _):
        for b in range(2):
            i = p * 2 + b
            base = base_t + i * CHL
            wait(b)

            @pl.when(p > 0)
            def _():
                wait_wb(b)
            av, dv, eb = avs[b], dvs[b], ebs[b]

            def grp(g, _):
                off = g * L + iota
                a16 = plsc.load_gather(av, [off])
                d16 = plsc.load_gather(dv, [off])
                mv = plsc.load_gather(mloc, [d16])
                e16 = jnp.exp(a16 - mv)
                plsc.store_scatter(eb, [off], e16)
                plsc.addupdate_scatter(ls, [d16], e16)
                return 0

            lax.fori_loop(0, CHL // L, grp, 0)
            pltpu.async_copy(ebs[b], e_hbm.at[pl.ds(base, CHL)], wsems[b])

            @pl.when(p * 2 + b + 2 < NCHL)
            def _():
                issue(i + 2, b)
        return 0

    lax.fori_loop(0, NCHL // 2, pair, 0)
    wait_wb(0)
    wait_wb(1)
    pltpu.sync_copy(ls, locsum_hbm.at[pl.ds(wid * SEG_P, SEG_P)])


def _expsum_call(a, dstp, m):
    kfn = pl.kernel(
        _expsum_body,
        out_type=(jax.ShapeDtypeStruct((EP,), f32),
                  jax.ShapeDtypeStruct((NW * SEG_P,), f32)),
        mesh=_mesh(),
        compiler_params=_SC_PARAMS,
        scratch_types=[
            pltpu.VMEM((CHL,), f32),
            pltpu.VMEM((CHL,), f32),
            pltpu.VMEM((CHL,), i32),
            pltpu.VMEM((CHL,), i32),
            pltpu.VMEM((CHL,), f32),
            pltpu.VMEM((CHL,), f32),
            pltpu.VMEM((SEG_P,), f32),
            pltpu.VMEM((SEG_P,), f32),
            pltpu.SemaphoreType.DMA,
            pltpu.SemaphoreType.DMA,
            pltpu.SemaphoreType.DMA,
            pltpu.SemaphoreType.DMA,
        ],
    )
    return kfn(a, dstp, m)


def _scatter_body(x0_hbm, x1_hbm, x2_hbm, x3_hbm, src_hbm, dst_hbm, e_hbm,
                  s_hbm, z_hbm, out_hbm,
                  si0, si1, si2, si3, dv0, dv1, dv2, dv3,
                  ev0, ev1, ev2, ev3, wb0, wb1, xr0, xr1, sloc, acc,
                  isem0, isem1, isem2, isem3, gsem0, gsem1, ssem0, ssem1):
    cid = lax.axis_index("c")
    sid = lax.axis_index("s")
    sis = (si0, si1, si2, si3)
    dvs = (dv0, dv1, dv2, dv3)
    evs = (ev0, ev1, ev2, ev3)
    wbs = (wb0, wb1)
    xrs = (xr0, xr1)
    isems = (isem0, isem1, isem2, isem3)
    gsems = (gsem0, gsem1)
    ssems = (ssem0, ssem1)
    base_t = sid * TPE_E
    NQ = NCH_E // 4

    pltpu.sync_copy(s_hbm, sloc)
    r0 = sid * ROWS_T

    def issue_idx(i, j):
        base = base_t + i * CH
        pltpu.async_copy(src_hbm.at[pl.ds(base, CH)], sis[j], isems[j])
        pltpu.async_copy(dst_hbm.at[pl.ds(base, CH)], dvs[j], isems[j])
        pltpu.async_copy(e_hbm.at[pl.ds(base, CH)], evs[j], isems[j])

    def wait_idx(j):
        pltpu.make_async_copy(src_hbm.at[pl.ds(0, CH)], sis[j],
                              isems[j]).wait()
        pltpu.make_async_copy(dst_hbm.at[pl.ds(0, CH)], dvs[j],
                              isems[j]).wait()
        pltpu.make_async_copy(e_hbm.at[pl.ds(0, CH)], evs[j],
                              isems[j]).wait()

    def wait_scat(b, j):
        pltpu.make_async_copy(xrs[b], acc.at[dvs[j]], ssems[b]).wait()

    for p in range(2):
        t0 = x0_hbm if p == 0 else x1_hbm
        t1 = x2_hbm if p == 0 else x3_hbm

        def issue_gather(b, j):
            @pl.when(cid == 0)
            def _():
                pltpu.async_copy(t0.at[sis[j]], xrs[b], gsems[b])

            @pl.when(cid != 0)
            def _():
                pltpu.async_copy(t1.at[sis[j]], xrs[b], gsems[b])

        def wait_gather(b, j):
            pltpu.make_async_copy(t0.at[sis[j]], xrs[b], gsems[b]).wait()

        pltpu.sync_copy(z_hbm.at[pl.ds(r0, ROWS_T)],
                        acc.at[pl.ds(r0, ROWS_T)])
        plsc.subcore_barrier()

        issue_idx(0, 0)
        issue_idx(1, 1)
        wait_idx(0)
        issue_gather(0, 0)

        def quad(q, _):
            for k in range(4):
                i = q * 4 + k
                b = k % 2
                bn = 1 - b
                jn = (k + 1) % 4

                def stage_next():
                    wait_idx(jn)

                    @pl.when(i >= 1)
                    def _():
                        wait_scat(bn, (k - 1) % 4)
                    issue_gather(bn, jn)

                if k == 3:
                    @pl.when(q < NQ - 1)
                    def _():
                        stage_next()
                else:
                    stage_next()

                wait_gather(b, k)

                for g in range(CH // L):
                    sl = pl.ds(g * L, L)
                    sv = plsc.load_gather(sloc, [dvs[k][sl]])
                    wbs[b][sl] = evs[k][sl] / sv

                xr = xrs[b]
                wb = wbs[b]

                def edge(e, _):
                    ws = plsc.load_gather(wb, [jnp.full((L,), 0, i32) + e])
                    for g in range(DQ // L):
                        sl = pl.ds(g * L, L)
                        xr[e, sl] = xr[e, sl] * ws
                    return 0

                lax.fori_loop(0, CH, edge, 0)
                pltpu.async_copy(xrs[b], acc.at[dvs[k]], ssems[b], add=True)

                if k < 2:
                    @pl.when(q * 4 + k + 2 < NCH_E)
                    def _():
                        issue_idx(i + 2, (k + 2) % 4)
                else:
                    @pl.when(q < NQ - 1)
                    def _():
                        issue_idx(i + 2, (k + 2) % 4)
            return 0

        lax.fori_loop(0, NQ, quad, 0)
        wait_scat(0, 2)
        wait_scat(1, 3)
        plsc.subcore_barrier()
        pltpu.sync_copy(acc.at[pl.ds(r0, ROWS_T)],
                        out_hbm.at[cid * 2 + p, pl.ds(r0, ROWS_T)])
        plsc.subcore_barrier()


def _scatter_call(xq, srcp, dstp, e, s, zeros_hbm):
    kfn = pl.kernel(
        _scatter_body,
        out_type=jax.ShapeDtypeStruct((4, SEG_P, DQ), f32),
        mesh=_mesh(),
        compiler_params=_SC_PARAMS_SCT,
        scratch_types=(
            [pltpu.VMEM((CH,), i32)] * 8
            + [pltpu.VMEM((CH,), f32)] * 6
            + [pltpu.VMEM((CH, DQ), f32)] * 2
            + [pltpu.VMEM((SEG_P,), f32),
               pltpu.VMEM_SHARED((SEG_P, DQ), f32)]
            + [pltpu.SemaphoreType.DMA] * 8
        ),
    )
    return kfn(xq[0], xq[1], xq[2], xq[3], srcp, dstp, e, s, zeros_hbm)


def _pad_edges(src, dst, npid):
    pad = EP - EDGES
    src = jnp.concatenate([src, jnp.zeros((pad,), i32)])
    npid = jnp.concatenate([npid, jnp.zeros((pad,), i32)])
    dst = jnp.concatenate([dst, jnp.full((pad,), PAD_DST, i32)])
    return src, dst, npid


def _etype(h, ehp, x, srcp, dstp, npidp, att_w, zeros_hbm):
    hh = _gather_call(h, ehp, srcp, npidp)
    a = _attdot(hh, att_w).reshape(EP)
    locmax = _locmax_call(a, dstp)
    m = _combine_call(locmax, True)
    e, locsum = _expsum_call(a, dstp, m)
    s = _combine_call(locsum, False)
    xq = [x[:, DQ * q:DQ * (q + 1)] + 0.0 for q in range(4)]
    out4 = _scatter_call(xq, srcp, dstp, e, s, zeros_hbm)
    return jnp.concatenate([out4[q, :N, :] for q in range(4)], axis=1)


def kernel(x_user, x_item, edge_index_ui, npid_ui, edge_index_iu, npid_iu,
           W_o, b_o, W_u, b_u, W_g, b_g, att_w, att_b, emb_user, emb_item):
    src_ui = edge_index_ui[0].astype(i32)
    dst_ui = edge_index_ui[1].astype(i32)
    src_iu = edge_index_iu[0].astype(i32)
    dst_iu = edge_index_iu[1].astype(i32)
    npid_ui = npid_ui.astype(i32)
    npid_iu = npid_iu.astype(i32)

    src_ui, dst_ui, npid_ui = _pad_edges(src_ui, dst_ui, npid_ui)
    src_iu, dst_iu, npid_iu = _pad_edges(src_iu, dst_iu, npid_iu)

    s_u = _colsum(x_user)
    s_i = _colsum(x_item)
    h_u, ehp_u = _proj(x_user, emb_user, s_u, W_o, b_o, W_g, b_g, W_u, b_u)
    h_i, ehp_i = _proj(x_item, emb_item, s_i, W_o, b_o, W_g, b_g, W_u, b_u)

    zeros_hbm = jnp.zeros((SEG_P, DQ), f32)

    out_item = _etype(h_u, ehp_u, x_user, src_ui, dst_ui, npid_ui, att_w,
                      zeros_hbm)
    out_user = _etype(h_i, ehp_i, x_item, src_iu, dst_iu, npid_iu, att_w,
                      zeros_hbm)
    return (out_user, out_item)


# speedup vs baseline: 1.1893x; 1.1893x over previous
import functools

import jax
import jax.numpy as jnp
from jax import lax
from jax.experimental import pallas as pl
from jax.experimental.pallas import tpu as pltpu
from jax.experimental.pallas import tpu_sc as plsc

N = 25000
D = 128
EDGES = 312500
L = 16
NC, NS = 2, 16
NW = NC * NS
SEG_P = 25088
SEG_T = SEG_P // NW
PAD_DST = 25024
TPE = 9984
EP = NW * TPE
CH = 128
NCH_A = TPE // CH
CHL = 1248
NCHL = TPE // CHL
TPE_E = EP // NS
NCH_E = TPE_E // CH
ROWS_T = SEG_P // NS
DH = D // 2
DQ = D // 4
NEG = -3.0e38

f32 = jnp.float32
i32 = jnp.int32


def _mesh():
    return plsc.VectorSubcoreMesh(core_axis_name="c", subcore_axis_name="s")


_SC_PARAMS = pltpu.CompilerParams(needs_layout_passes=False)
_SC_PARAMS_SCT = pltpu.CompilerParams(needs_layout_passes=False,
                                      use_tc_tiling_on_sc=False)


def _colsum(x):
    blk = 1000
    nblk = N // blk

    def k(x_ref, o_ref):
        i = pl.program_id(0)
        part = jnp.sum(x_ref[...], axis=0, keepdims=True)
        pb = jnp.broadcast_to(part, (8, D))

        @pl.when(i == 0)
        def _():
            o_ref[...] = pb

        @pl.when(i != 0)
        def _():
            o_ref[...] = o_ref[...] + pb

    return pl.pallas_call(
        k,
        grid=(nblk,),
        in_specs=[pl.BlockSpec((blk, D), lambda i: (i, 0))],
        out_specs=pl.BlockSpec((8, D), lambda i: (0, 0)),
        out_shape=jax.ShapeDtypeStruct((8, D), f32),
    )(x)


def _proj(x, emb, s8, wo, bo, wg, bg, wu, bu):
    blk = 1000
    nblk = N // blk

    def k(x_ref, emb_ref, s_ref, wo_ref, bo_ref, wg_ref, bg_ref, wu_ref,
          bu_ref, h_ref, ehp_ref):
        s = s_ref[0:1, :]
        g = (jnp.dot(s, wg_ref[...], preferred_element_type=f32)
             + bg_ref[...] + bo_ref[...])
        h_ref[...] = jnp.dot(x_ref[...], wo_ref[...],
                             preferred_element_type=f32) + g
        ehp_ref[...] = jnp.dot(emb_ref[...], wu_ref[...],
                               preferred_element_type=f32) + bu_ref[...]

    full = lambda shape: pl.BlockSpec(shape, lambda i: (0,) * len(shape))
    return pl.pallas_call(
        k,
        grid=(nblk,),
        in_specs=[
            pl.BlockSpec((blk, D), lambda i: (i, 0)),
            pl.BlockSpec((blk, D), lambda i: (i, 0)),
            full((8, D)),
            full((D, D)), full((1, D)),
            full((D, D)), full((1, D)),
            full((D, D)), full((1, D)),
        ],
        out_specs=[
            pl.BlockSpec((blk, D), lambda i: (i, 0)),
            pl.BlockSpec((blk, D), lambda i: (i, 0)),
        ],
        out_shape=[
            jax.ShapeDtypeStruct((N, D), f32),
            jax.ShapeDtypeStruct((N, D), f32),
        ],
    )(x, emb, s8, wo, bo.reshape(1, D), wg, bg.reshape(1, D), wu,
      bu.reshape(1, D))


def _rne(x):
    u = plsc.bitcast(x, jnp.uint32)
    lsb = jax.lax.shift_right_logical(u, jnp.uint32(16)) & jnp.uint32(1)
    u2 = (u + jnp.uint32(0x7FFF) + lsb) & jnp.uint32(0xFFFF0000)
    return plsc.bitcast(u2, f32)


def _rnd2(x, y):
    return _rne(x), _rne(y)


def _attn_body(h_hbm, ehp_hbm, src_hbm, npid_hbm, dst_hbm, aw_hbm,
               a_hbm, locmax_hbm,
               hb0, hb1, eb0, eb1, si0, si1, ni0, ni1, dv0, dv1,
               ab0, ab1, awv, lm,
               isem0, isem1, gsem0, gsem1, wsem0, wsem1):
    wid = lax.axis_index("s") * NC + lax.axis_index("c")
    base_t = wid * TPE
    iota = lax.iota(i32, L)
    masks = [iota == l for l in range(L)]
    hbs, ebs, abs_ = (hb0, hb1), (eb0, eb1), (ab0, ab1)
    sis, nis, dvs = (si0, si1), (ni0, ni1), (dv0, dv1)
    isems, gsems, wsems = (isem0, isem1), (gsem0, gsem1), (wsem0, wsem1)

    pltpu.sync_copy(aw_hbm, awv)
    wvr = []
    for gp in range(D // L // 2):
        w0 = awv[pl.ds((2 * gp) * L, L)]
        w1 = awv[pl.ds((2 * gp + 1) * L, L)]
        r0, r1 = _rnd2(w0, w1)
        wvr.extend([r0, r1])

    def issue_idx(i, b):
        base = base_t + i * CH
        pltpu.async_copy(src_hbm.at[pl.ds(base, CH)], sis[b], isems[b])
        pltpu.async_copy(npid_hbm.at[pl.ds(base, CH)], nis[b], isems[b])
        pltpu.async_copy(dst_hbm.at[pl.ds(base, CH)], dvs[b], isems[b])

    def wait_idx(b):
        pltpu.make_async_copy(src_hbm.at[pl.ds(0, CH)], sis[b],
                              isems[b]).wait()
        pltpu.make_async_copy(npid_hbm.at[pl.ds(0, CH)], nis[b],
                              isems[b]).wait()
        pltpu.make_async_copy(dst_hbm.at[pl.ds(0, CH)], dvs[b],
                              isems[b]).wait()

    def issue_gather(b):
        pltpu.async_copy(h_hbm.at[sis[b]], hbs[b], gsems[b])
        pltpu.async_copy(ehp_hbm.at[nis[b]], ebs[b], gsems[b])

    def wait_gather(b):
        pltpu.make_async_copy(h_hbm.at[sis[b]], hbs[b], gsems[b]).wait()
        pltpu.make_async_copy(ehp_hbm.at[nis[b]], ebs[b], gsems[b]).wait()

    def wait_wb(b):
        pltpu.make_async_copy(abs_[b], a_hbm.at[pl.ds(base_t, CH)],
                              wsems[b]).wait()

    neg = jnp.full((L,), NEG, f32)

    def init(iblk, _):
        plsc.store_scatter(lm, [iblk * L + iota], neg)
        return 0

    issue_idx(0, 0)
    issue_idx(1, 1)
    lax.fori_loop(0, SEG_P // L, init, 0)
    wait_idx(0)
    issue_gather(0)

    NP = NCH_A // 2

    def pair(p, _):
        for b in range(2):
            i = p * 2 + b
            base = base_t + i * CH
            bn = 1 - b
            if b == 0:
                wait_idx(bn)
                issue_gather(bn)
            else:
                @pl.when(p < NP - 1)
                def _():
                    wait_idx(bn)
                    issue_gather(bn)
            wait_gather(b)

            @pl.when(p < NP - 1)
            def _():
                issue_idx(i + 2, b)

            @pl.when(i >= 2)
            def _():
                wait_wb(b)

            hb, eb, ab, dv = hbs[b], ebs[b], abs_[b], dvs[b]

            def group(g, _):
                a16 = jnp.zeros((L,), f32)
                for k in range(L):
                    e = g * L + k
                    lvs = []
                    for q in range(D // L):
                        sl = pl.ds(q * L, L)
                        v = hb[e, sl] + eb[e, sl]
                        lvs.append(jnp.maximum(v, v * 0.01))
                    acc = jnp.zeros((L,), f32)
                    for gp in range(D // L // 2):
                        r0, r1 = _rnd2(lvs[2 * gp], lvs[2 * gp + 1])
                        acc = (acc + r0 * wvr[2 * gp]
                               + r1 * wvr[2 * gp + 1])
                    a16 = jnp.where(masks[k], jnp.sum(acc), a16)
                plsc.store_scatter(ab, [g * L + iota], a16)
                d16 = plsc.load_gather(dv, [g * L + iota])
                for l in range(L):
                    cur = plsc.load_gather(lm, [d16])
                    plsc.store_scatter(lm, [d16], jnp.maximum(cur, a16),
                                       mask=masks[l])
                return 0

            lax.fori_loop(0, CH // L, group, 0)
            pltpu.async_copy(abs_[b], a_hbm.at[pl.ds(base, CH)], wsems[b])
        return 0

    lax.fori_loop(0, NP, pair, 0)
    wait_wb(0)
    wait_wb(1)
    pltpu.sync_copy(lm, locmax_hbm.at[pl.ds(wid * SEG_P, SEG_P)])


def _attn_call(h, ehp, srcp, npidp, dstp, awf):
    kfn = pl.kernel(
        _attn_body,
        out_type=(jax.ShapeDtypeStruct((EP,), f32),
                  jax.ShapeDtypeStruct((NW * SEG_P,), f32)),
        mesh=_mesh(),
        compiler_params=_SC_PARAMS,
        scratch_types=(
            [pltpu.VMEM((CH, D), f32)] * 4
            + [pltpu.VMEM((CH,), i32)] * 6
            + [pltpu.VMEM((CH,), f32)] * 2
            + [pltpu.VMEM((D,), f32), pltpu.VMEM((SEG_P,), f32)]
            + [pltpu.SemaphoreType.DMA] * 6
        ),
    )
    return kfn(h, ehp, srcp, npidp, dstp, awf)


def _combine_body(is_max, loc_hbm, out_hbm, acc, buf0, buf1, sem0, sem1):
    wid = lax.axis_index("s") * NC + lax.axis_index("c")
    seg0 = wid * SEG_T
    bufs, sems = (buf0, buf1), (sem0, sem1)

    def issue(r, b):
        pltpu.async_copy(loc_hbm.at[pl.ds(r * SEG_P + seg0, SEG_T)],
                         bufs[b], sems[b])

    def wait(b):
        pltpu.make_async_copy(loc_hbm.at[pl.ds(seg0, SEG_T)], bufs[b],
                              sems[b]).wait()

    issue(0, 0)
    issue(1, 1)
    fill = jnp.full((L,), NEG if is_max else 0.0, f32)
    for ib in range(SEG_T // L):
        acc[pl.ds(ib * L, L)] = fill

    def pair(p, _):
        for b in range(2):
            r = p * 2 + b
            wait(b)
            for ib in range(SEG_T // L):
                sl = pl.ds(ib * L, L)
                x = acc[sl]
                y = bufs[b][sl]
                acc[sl] = jnp.maximum(x, y) if is_max else x + y

            @pl.when(r + 2 < NW)
            def _():
                issue(r + 2, b)
        return 0

    lax.fori_loop(0, NW // 2, pair, 0)
    pltpu.sync_copy(acc, out_hbm.at[pl.ds(seg0, SEG_T)])


def _combine_call(loc, is_max):
    kfn = pl.kernel(
        functools.partial(_combine_body, is_max),
        out_type=jax.ShapeDtypeStruct((SEG_P,), f32),
        mesh=_mesh(),
        compiler_params=_SC_PARAMS,
        scratch_types=[
            pltpu.VMEM((SEG_T,), f32),
            pltpu.VMEM((SEG_T,), f32),
            pltpu.VMEM((SEG_T,), f32),
            pltpu.SemaphoreType.DMA,
            pltpu.SemaphoreType.DMA,
        ],
    )
    return kfn(loc)


def _expsum_body(a_hbm, dst_hbm, m_hbm, e_hbm, locsum_hbm,
                 av0, av1, dv0, dv1, eb0, eb1, mloc, ls,
                 isem0, isem1, wsem0, wsem1):
    wid = lax.axis_index("s") * NC + lax.axis_index("c")
    base_t = wid * TPE
    iota = lax.iota(i32, L)
    masks = [iota == l for l in range(L)]
    avs, dvs, ebs = (av0, av1), (dv0, dv1), (eb0, eb1)
    isems, wsems = (isem0, isem1), (wsem0, wsem1)

    def issue(i, b):
        base = base_t + i * CHL
        pltpu.async_copy(a_hbm.at[pl.ds(base, CHL)], avs[b], isems[b])
        pltpu.async_copy(dst_hbm.at[pl.ds(base, CHL)], dvs[b], isems[b])

    def wait(b):
        pltpu.make_async_copy(a_hbm.at[pl.ds(0, CHL)], avs[b],
                              isems[b]).wait()
        pltpu.make_async_copy(dst_hbm.at[pl.ds(0, CHL)], dvs[b],
                              isems[b]).wait()

    def wait_wb(b):
        pltpu.make_async_copy(ebs[b], e_hbm.at[pl.ds(base_t, CHL)],
                              wsems[b]).wait()

    issue(0, 0)
    issue(1, 1)
    pltpu.sync_copy(m_hbm, mloc)

    zero = jnp.zeros((L,), f32)

    def init(iblk, _):
        plsc.store_scatter(ls, [iblk * L + iota], zero)
        return 0

    lax.fori_loop(0, SEG_P // L, init, 0)

    def pair(p, _):
        for b in range(2):
            i = p * 2 + b
            base = base_t + i * CHL
            wait(b)

            @pl.when(p > 0)
            def _():
                wait_wb(b)
            av, dv, eb = avs[b], dvs[b], ebs[b]

            def grp(g, _):
                off = g * L + iota
                a16 = plsc.load_gather(av, [off])
                d16 = plsc.load_gather(dv, [off])
                mv = plsc.load_gather(mloc, [d16])
                e16 = jnp.exp(a16 - mv)
                plsc.store_scatter(eb, [off], e16)
                plsc.addupdate_scatter(ls, [d16], e16)
                return 0

            lax.fori_loop(0, CHL // L, grp, 0)
            pltpu.async_copy(ebs[b], e_hbm.at[pl.ds(base, CHL)], wsems[b])

            @pl.when(p * 2 + b + 2 < NCHL)
            def _():
                issue(i + 2, b)
        return 0

    lax.fori_loop(0, NCHL // 2, pair, 0)
    wait_wb(0)
    wait_wb(1)
    pltpu.sync_copy(ls, locsum_hbm.at[pl.ds(wid * SEG_P, SEG_P)])


def _expsum_call(a, dstp, m):
    kfn = pl.kernel(
        _expsum_body,
        out_type=(jax.ShapeDtypeStruct((EP,), f32),
                  jax.ShapeDtypeStruct((NW * SEG_P,), f32)),
        mesh=_mesh(),
        compiler_params=_SC_PARAMS,
        scratch_types=[
            pltpu.VMEM((CHL,), f32),
            pltpu.VMEM((CHL,), f32),
            pltpu.VMEM((CHL,), i32),
            pltpu.VMEM((CHL,), i32),
            pltpu.VMEM((CHL,), f32),
            pltpu.VMEM((CHL,), f32),
            pltpu.VMEM((SEG_P,), f32),
            pltpu.VMEM((SEG_P,), f32),
            pltpu.SemaphoreType.DMA,
            pltpu.SemaphoreType.DMA,
            pltpu.SemaphoreType.DMA,
            pltpu.SemaphoreType.DMA,
        ],
    )
    return kfn(a, dstp, m)


def _scatter_body(x0_hbm, x1_hbm, x2_hbm, x3_hbm, src_hbm, dst_hbm, e_hbm,
                  s_hbm, z_hbm, out_hbm,
                  si0, si1, si2, si3, dv0, dv1, dv2, dv3,
                  ev0, ev1, ev2, ev3, wb0, wb1, xr0, xr1, sloc, acc,
                  isem0, isem1, isem2, isem3, gsem0, gsem1, ssem0, ssem1):
    cid = lax.axis_index("c")
    sid = lax.axis_index("s")
    sis = (si0, si1, si2, si3)
    dvs = (dv0, dv1, dv2, dv3)
    evs = (ev0, ev1, ev2, ev3)
    wbs = (wb0, wb1)
    xrs = (xr0, xr1)
    isems = (isem0, isem1, isem2, isem3)
    gsems = (gsem0, gsem1)
    ssems = (ssem0, ssem1)
    base_t = sid * TPE_E
    NQ = NCH_E // 4

    pltpu.sync_copy(s_hbm, sloc)
    r0 = sid * ROWS_T

    def issue_idx(i, j):
        base = base_t + i * CH
        pltpu.async_copy(src_hbm.at[pl.ds(base, CH)], sis[j], isems[j])
        pltpu.async_copy(dst_hbm.at[pl.ds(base, CH)], dvs[j], isems[j])
        pltpu.async_copy(e_hbm.at[pl.ds(base, CH)], evs[j], isems[j])

    def wait_idx(j):
        pltpu.make_async_copy(src_hbm.at[pl.ds(0, CH)], sis[j],
                              isems[j]).wait()
        pltpu.make_async_copy(dst_hbm.at[pl.ds(0, CH)], dvs[j],
                              isems[j]).wait()
        pltpu.make_async_copy(e_hbm.at[pl.ds(0, CH)], evs[j],
                              isems[j]).wait()

    def wait_scat(b, j):
        pltpu.make_async_copy(xrs[b], acc.at[dvs[j]], ssems[b]).wait()

    for p in range(2):
        t0 = x0_hbm if p == 0 else x1_hbm
        t1 = x2_hbm if p == 0 else x3_hbm

        def issue_gather(b, j):
            @pl.when(cid == 0)
            def _():
                pltpu.async_copy(t0.at[sis[j]], xrs[b], gsems[b])

            @pl.when(cid != 0)
            def _():
                pltpu.async_copy(t1.at[sis[j]], xrs[b], gsems[b])

        def wait_gather(b, j):
            pltpu.make_async_copy(t0.at[sis[j]], xrs[b], gsems[b]).wait()

        pltpu.sync_copy(z_hbm.at[pl.ds(r0, ROWS_T)],
                        acc.at[pl.ds(r0, ROWS_T)])
        plsc.subcore_barrier()

        issue_idx(0, 0)
        issue_idx(1, 1)
        wait_idx(0)
        issue_gather(0, 0)

        def quad(q, _):
            for k in range(4):
                i = q * 4 + k
                b = k % 2
                bn = 1 - b
                jn = (k + 1) % 4

                def stage_next():
                    wait_idx(jn)

                    @pl.when(i >= 1)
                    def _():
                        wait_scat(bn, (k - 1) % 4)
                    issue_gather(bn, jn)

                if k == 3:
                    @pl.when(q < NQ - 1)
                    def _():
                        stage_next()
                else:
                    stage_next()

                wait_gather(b, k)

                for g in range(CH // L):
                    sl = pl.ds(g * L, L)
                    sv = plsc.load_gather(sloc, [dvs[k][sl]])
                    wbs[b][sl] = evs[k][sl] / sv

                xr = xrs[b]
                wb = wbs[b]

                def edge(e, _):
                    ws = plsc.load_gather(wb, [jnp.full((L,), 0, i32) + e])
                    for g in range(DQ // L):
                        sl = pl.ds(g * L, L)
                        xr[e, sl] = xr[e, sl] * ws
                    return 0

                lax.fori_loop(0, CH, edge, 0)
                pltpu.async_copy(xrs[b], acc.at[dvs[k]], ssems[b], add=True)

                if k < 2:
                    @pl.when(q * 4 + k + 2 < NCH_E)
                    def _():
                        issue_idx(i + 2, (k + 2) % 4)
                else:
                    @pl.when(q < NQ - 1)
                    def _():
                        issue_idx(i + 2, (k + 2) % 4)
            return 0

        lax.fori_loop(0, NQ, quad, 0)
        wait_scat(0, 2)
        wait_scat(1, 3)
        plsc.subcore_barrier()
        pltpu.sync_copy(acc.at[pl.ds(r0, ROWS_T)],
                        out_hbm.at[cid * 2 + p, pl.ds(r0, ROWS_T)])
        plsc.subcore_barrier()


def _scatter_call(xq, srcp, dstp, e, s, zeros_hbm):
    kfn = pl.kernel(
        _scatter_body,
        out_type=jax.ShapeDtypeStruct((4, SEG_P, DQ), f32),
        mesh=_mesh(),
        compiler_params=_SC_PARAMS_SCT,
        scratch_types=(
            [pltpu.VMEM((CH,), i32)] * 8
            + [pltpu.VMEM((CH,), f32)] * 6
            + [pltpu.VMEM((CH, DQ), f32)] * 2
            + [pltpu.VMEM((SEG_P,), f32),
               pltpu.VMEM_SHARED((SEG_P, DQ), f32)]
            + [pltpu.SemaphoreType.DMA] * 8
        ),
    )
    return kfn(xq[0], xq[1], xq[2], xq[3], srcp, dstp, e, s, zeros_hbm)


def _pad_edges(src, dst, npid):
    pad = EP - EDGES
    src = jnp.concatenate([src, jnp.zeros((pad,), i32)])
    npid = jnp.concatenate([npid, jnp.zeros((pad,), i32)])
    dst = jnp.concatenate([dst, jnp.full((pad,), PAD_DST, i32)])
    return src, dst, npid


def _etype(h, ehp, x, srcp, dstp, npidp, att_w, zeros_hbm):
    a, locmax = _attn_call(h, ehp, srcp, npidp, dstp, att_w[:, 0])
    m = _combine_call(locmax, True)
    e, locsum = _expsum_call(a, dstp, m)
    s = _combine_call(locsum, False)
    xq = [x[:, DQ * q:DQ * (q + 1)] + 0.0 for q in range(4)]
    out4 = _scatter_call(xq, srcp, dstp, e, s, zeros_hbm)
    return jnp.concatenate([out4[q, :N, :] for q in range(4)], axis=1)


def kernel(x_user, x_item, edge_index_ui, npid_ui, edge_index_iu, npid_iu,
           W_o, b_o, W_u, b_u, W_g, b_g, att_w, att_b, emb_user, emb_item):
    src_ui = edge_index_ui[0].astype(i32)
    dst_ui = edge_index_ui[1].astype(i32)
    src_iu = edge_index_iu[0].astype(i32)
    dst_iu = edge_index_iu[1].astype(i32)
    npid_ui = npid_ui.astype(i32)
    npid_iu = npid_iu.astype(i32)

    src_ui, dst_ui, npid_ui = _pad_edges(src_ui, dst_ui, npid_ui)
    src_iu, dst_iu, npid_iu = _pad_edges(src_iu, dst_iu, npid_iu)

    s_u = _colsum(x_user)
    s_i = _colsum(x_item)
    h_u, ehp_u = _proj(x_user, emb_user, s_u, W_o, b_o, W_g, b_g, W_u, b_u)
    h_i, ehp_i = _proj(x_item, emb_item, s_i, W_o, b_o, W_g, b_g, W_u, b_u)

    zeros_hbm = jnp.zeros((SEG_P, DQ), f32)

    out_item = _etype(h_u, ehp_u, x_user, src_ui, dst_ui, npid_ui, att_w,
                      zeros_hbm)
    out_user = _etype(h_i, ehp_i, x_item, src_iu, dst_iu, npid_iu, att_w,
                      zeros_hbm)
    return (out_user, out_item)
